# Initial kernel scaffold; baseline (speedup 1.0000x reference)
import jax
import jax.numpy as jnp
from jax.experimental import pallas as pl


def kernel(x, W, b, space_table, nan_table, pe):
    raise NotImplementedError("write your pallas kernel here")


# speedup vs baseline: 6.9764x; 6.9764x over previous
import jax
import jax.numpy as jnp
from jax.experimental import pallas as pl
from jax.experimental.pallas import tpu as pltpu


def _body(x_ref, wt_ref, b_ref, pe_ref, sp_ref, nan_ref, out_ref, base_ref):
    ntok, d_model = base_ref.shape
    t_steps = pe_ref.shape[0]
    n_sp = sp_ref.shape[0]

    @pl.when(pl.program_id(0) == 0)
    def _():
        ri = jax.lax.broadcasted_iota(jnp.int32, (ntok, t_steps), 0) // n_sp
        ci = jax.lax.broadcasted_iota(jnp.int32, (ntok, t_steps), 1)
        rt = (ri == ci).astype(jnp.float32)
        si = jax.lax.broadcasted_iota(jnp.int32, (ntok, n_sp), 0) % n_sp
        cj = jax.lax.broadcasted_iota(jnp.int32, (ntok, n_sp), 1)
        rs = (si == cj).astype(jnp.float32)
        base = jnp.dot(rt, pe_ref[...], preferred_element_type=jnp.float32)
        base = base + jnp.dot(rs, sp_ref[...], preferred_element_type=jnp.float32)
        base_ref[...] = base + b_ref[...]

    xb = x_ref[...]
    m = jnp.isnan(xb)
    xc = jnp.where(m, 0.0, xb)
    mask = jnp.any(m, axis=-1, keepdims=True)
    wt = wt_ref[...]
    w0 = wt[0:1, :].reshape(1, 1, d_model)
    w1 = wt[1:2, :].reshape(1, 1, d_model)
    w2 = wt[2:3, :].reshape(1, 1, d_model)
    xl = xc[:, :, 0:1] * w0 + xc[:, :, 1:2] * w1 + xc[:, :, 2:3] * w2
    n0 = nan_ref[0:1, :].reshape(1, 1, d_model)
    n1 = nan_ref[1:2, :].reshape(1, 1, d_model)
    nanem = jnp.where(mask, n1, n0)
    out_ref[...] = xl + nanem + base_ref[...][None]


def kernel(x, W, b, space_table, nan_table, pe):
    bsize = x.shape[0]
    d_x = W.shape[1]
    d_model = W.shape[0]
    xr = x.reshape(bsize, -1, d_x)
    ntok = xr.shape[1]

    bb = 4
    grid = (bsize // bb,)
    out = pl.pallas_call(
        _body,
        grid=grid,
        in_specs=[
            pl.BlockSpec((bb, ntok, d_x), lambda i: (i, 0, 0)),
            pl.BlockSpec((d_x, d_model), lambda i: (0, 0)),
            pl.BlockSpec((1, d_model), lambda i: (0, 0)),
            pl.BlockSpec(pe.shape, lambda i: (0, 0)),
            pl.BlockSpec(space_table.shape, lambda i: (0, 0)),
            pl.BlockSpec(nan_table.shape, lambda i: (0, 0)),
        ],
        out_specs=pl.BlockSpec((bb, ntok, d_model), lambda i: (i, 0, 0)),
        out_shape=jax.ShapeDtypeStruct((bsize, ntok, d_model), jnp.float32),
        scratch_shapes=[pltpu.VMEM((ntok, d_model), jnp.float32)],
    )(xr, W.T, b.reshape(1, -1), pe, space_table, nan_table)
    return out


# speedup vs baseline: 8.2431x; 1.1816x over previous
import jax
import jax.numpy as jnp
from jax.experimental import pallas as pl
from jax.experimental.pallas import tpu as pltpu


def _body(x_ref, wt4_ref, bn_ref, pe_ref, sp_ref, out_ref, base_ref):
    ntok, d_model = base_ref.shape
    t_steps = pe_ref.shape[0]
    n_sp = sp_ref.shape[0]
    bb = x_ref.shape[0]

    @pl.when(pl.program_id(0) == 0)
    def _():
        ri = jax.lax.broadcasted_iota(jnp.int32, (ntok, t_steps), 0) // n_sp
        ci = jax.lax.broadcasted_iota(jnp.int32, (ntok, t_steps), 1)
        rt = (ri == ci).astype(jnp.float32)
        si = jax.lax.broadcasted_iota(jnp.int32, (ntok, n_sp), 0) % n_sp
        cj = jax.lax.broadcasted_iota(jnp.int32, (ntok, n_sp), 1)
        rs = (si == cj).astype(jnp.float32)
        base = jnp.dot(rt, pe_ref[...], preferred_element_type=jnp.float32)
        base = base + jnp.dot(rs, sp_ref[...], preferred_element_type=jnp.float32)
        base_ref[...] = base + bn_ref[...]

    xb = x_ref[...]
    m3 = jnp.isnan(xb)
    xc = jnp.where(m3, 0.0, xb)
    maskf = jnp.max(m3.astype(jnp.float32), axis=-1, keepdims=True)
    xin = jnp.concatenate([xc, maskf], axis=-1)
    base = base_ref[...]
    wt4 = wt4_ref[...]
    for i in range(bb):
        out_ref[i] = jnp.dot(xin[i], wt4, preferred_element_type=jnp.float32) + base


def kernel(x, W, b, space_table, nan_table, pe):
    bsize = x.shape[0]
    d_x = W.shape[1]
    d_model = W.shape[0]
    xr = x.reshape(bsize, -1, d_x)
    ntok = xr.shape[1]

    wt4 = jnp.concatenate([W.T, (nan_table[1] - nan_table[0])[None, :]], axis=0)
    bn = (b + nan_table[0]).reshape(1, -1)

    bb = 4
    grid = (bsize // bb,)
    out = pl.pallas_call(
        _body,
        grid=grid,
        in_specs=[
            pl.BlockSpec((bb, ntok, d_x), lambda i: (i, 0, 0)),
            pl.BlockSpec((d_x + 1, d_model), lambda i: (0, 0)),
            pl.BlockSpec((1, d_model), lambda i: (0, 0)),
            pl.BlockSpec(pe.shape, lambda i: (0, 0)),
            pl.BlockSpec(space_table.shape, lambda i: (0, 0)),
        ],
        out_specs=pl.BlockSpec((bb, ntok, d_model), lambda i: (i, 0, 0)),
        out_shape=jax.ShapeDtypeStruct((bsize, ntok, d_model), jnp.float32),
        scratch_shapes=[pltpu.VMEM((ntok, d_model), jnp.float32)],
    )(xr, wt4, bn, pe, space_table)
    return out


# speedup vs baseline: 8.5806x; 1.0409x over previous
import jax
import jax.numpy as jnp
from jax.experimental import pallas as pl
from jax.experimental.pallas import tpu as pltpu


def _body(x_ref, wt4_ref, bn_ref, pe_ref, sp_ref, out_ref, base_ref):
    ntok, d_model = base_ref.shape
    t_steps = pe_ref.shape[0]
    n_sp = sp_ref.shape[0]
    bb = x_ref.shape[0]

    @pl.when(pl.program_id(0) == 0)
    def _():
        ri = jax.lax.broadcasted_iota(jnp.int32, (ntok, t_steps), 0) // n_sp
        ci = jax.lax.broadcasted_iota(jnp.int32, (ntok, t_steps), 1)
        rt = (ri == ci).astype(jnp.float32)
        si = jax.lax.broadcasted_iota(jnp.int32, (ntok, n_sp), 0) % n_sp
        cj = jax.lax.broadcasted_iota(jnp.int32, (ntok, n_sp), 1)
        rs = (si == cj).astype(jnp.float32)
        base = jnp.dot(rt, pe_ref[...], preferred_element_type=jnp.float32)
        base = base + jnp.dot(rs, sp_ref[...], preferred_element_type=jnp.float32)
        base_ref[...] = base + bn_ref[...]

    xb = x_ref[...]
    m3 = jnp.isnan(xb)
    xc = jnp.where(m3, 0.0, xb)
    maskf = jnp.max(m3.astype(jnp.float32), axis=-1, keepdims=True)
    xin = jnp.concatenate([xc, maskf], axis=-1)
    base = base_ref[...]
    wt4 = wt4_ref[...]
    for i in range(bb):
        out_ref[i] = jnp.dot(xin[i], wt4, preferred_element_type=jnp.float32) + base


def kernel(x, W, b, space_table, nan_table, pe):
    bsize = x.shape[0]
    d_x = W.shape[1]
    d_model = W.shape[0]
    xr = x.reshape(bsize, -1, d_x)
    ntok = xr.shape[1]

    wt4 = jnp.concatenate([W.T, (nan_table[1] - nan_table[0])[None, :]], axis=0)
    bn = (b + nan_table[0]).reshape(1, -1)

    bb = 16
    grid = (bsize // bb,)
    out = pl.pallas_call(
        _body,
        grid=grid,
        in_specs=[
            pl.BlockSpec((bb, ntok, d_x), lambda i: (i, 0, 0)),
            pl.BlockSpec((d_x + 1, d_model), lambda i: (0, 0)),
            pl.BlockSpec((1, d_model), lambda i: (0, 0)),
            pl.BlockSpec(pe.shape, lambda i: (0, 0)),
            pl.BlockSpec(space_table.shape, lambda i: (0, 0)),
        ],
        out_specs=pl.BlockSpec((bb, ntok, d_model), lambda i: (i, 0, 0)),
        out_shape=jax.ShapeDtypeStruct((bsize, ntok, d_model), jnp.float32),
        scratch_shapes=[pltpu.VMEM((ntok, d_model), jnp.float32)],
    )(xr, wt4, bn, pe, space_table)
    return out
